# Initial kernel scaffold; baseline (speedup 1.0000x reference)
import jax
import jax.numpy as jnp
from jax.experimental import pallas as pl


def kernel(x, token_table, pos_table):
    raise NotImplementedError("write your pallas kernel here")


# speedup vs baseline: 1.4863x; 1.4863x over previous
import functools

import jax
import jax.numpy as jnp
from jax import lax
from jax.experimental import pallas as pl
from jax.experimental.pallas import tpu as pltpu
from jax.experimental.pallas import tpu_sc as plsc

B = 4096
S = 200
D = 32
NROWS = B * S

_info = plsc.get_sparse_core_info()
NC = _info.num_cores
NS = _info.num_subcores
NW = NC * NS

CB = 8
CR = CB * S
NCH = B // NW // CB
GSZ = 80
NG = CR // GSZ
NBUF = 2

assert B % (NW * CB) == 0 and CR % GSZ == 0


def _sc_body(x_hbm, tok_hbm, pos_hbm, out_hbm,
             idx0, idx1, rows0, rows1, pos_v,
             gsem0, gsem1, osem0, osem1):
  cid = lax.axis_index("c")
  sid = lax.axis_index("s")
  wid = sid * NC + cid

  idxs = (idx0, idx1)
  rows = (rows0, rows1)
  gsems = (gsem0, gsem1)
  osems = (osem0, osem1)

  pltpu.sync_copy(pos_hbm, pos_v)

  def fire_gathers(c, b):
    chunk = wid * NCH + c
    pltpu.sync_copy(x_hbm.at[chunk], idxs[b])
    for g in range(NG):
      pltpu.async_copy(tok_hbm.at[idxs[b].at[g]],
                       rows[b].at[pl.ds(g * GSZ, GSZ)], gsems[b])

  def drain_gathers(c, b):
    chunk = wid * NCH + c
    pltpu.make_async_copy(out_hbm.at[pl.ds(chunk * CR, CR)],
                          rows[b], gsems[b]).wait()

  def add_pos(b):
    def pbody(p, carry):
      pv0 = pos_v[p, pl.ds(0, 16)]
      pv1 = pos_v[p, pl.ds(16, 16)]
      for k in range(CB):
        plsc.addupdate(rows[b].at[p + S * k, pl.ds(0, 16)], pv0)
        plsc.addupdate(rows[b].at[p + S * k, pl.ds(16, 16)], pv1)
      return carry
    lax.fori_loop(0, S, pbody, 0)

  def fire_out(c, b):
    chunk = wid * NCH + c
    pltpu.async_copy(rows[b], out_hbm.at[pl.ds(chunk * CR, CR)], osems[b])

  def wait_out(c, b):
    chunk = wid * NCH + c
    pltpu.make_async_copy(rows[b], out_hbm.at[pl.ds(chunk * CR, CR)],
                          osems[b]).wait()

  fire_gathers(0, 0)
  fire_gathers(1, 1)

  def cbody(cc, carry):
    for b in range(NBUF):
      c = NBUF * cc + b
      drain_gathers(c, b)
      add_pos(b)
      fire_out(c, b)

    @pl.when(cc < NCH // NBUF - 1)
    def _refill():
      for b in range(NBUF):
        c = NBUF * cc + b
        wait_out(c, b)
        fire_gathers(c + NBUF, b)
    return carry

  lax.fori_loop(0, NCH // NBUF, cbody, 0)

  wait_out(NCH - NBUF, 0)
  wait_out(NCH - 1, 1)


_sc_embed = functools.partial(
    pl.kernel,
    out_type=jax.ShapeDtypeStruct((NROWS, D), jnp.float32),
    mesh=plsc.VectorSubcoreMesh(core_axis_name="c", subcore_axis_name="s"),
    compiler_params=pltpu.CompilerParams(use_tc_tiling_on_sc=False),
    scratch_types=[
        pltpu.VMEM((NG, GSZ), jnp.int32),
        pltpu.VMEM((NG, GSZ), jnp.int32),
        pltpu.VMEM((CR, D), jnp.float32),
        pltpu.VMEM((CR, D), jnp.float32),
        pltpu.VMEM((S, D), jnp.float32),
        pltpu.SemaphoreType.DMA,
        pltpu.SemaphoreType.DMA,
        pltpu.SemaphoreType.DMA,
        pltpu.SemaphoreType.DMA,
    ],
)(_sc_body)


@jax.jit
def kernel(x, token_table, pos_table):
  xr = x.astype(jnp.int32).reshape(NW * NCH, NG, GSZ)
  out = _sc_embed(xr, token_table, pos_table)
  return out.reshape(B, S, D)
